# speedup vs baseline: 1.4677x; 1.4677x over previous
import functools

import jax
import jax.numpy as jnp
import numpy as np
from jax.experimental import pallas as pl

_ANCHORS = np.array(
    [[12, 16], [19, 36], [40, 28], [36, 75], [76, 55], [72, 146],
     [142, 110], [192, 243], [459, 401]], dtype=np.float32)
_ANCHOR_MASKS = [[6, 7, 8], [3, 4, 5], [0, 1, 2]]
_DOWNSAMPLE = [32.0, 16.0, 8.0]
_OC = 85


def _decode_body(x_ref, o_ref, *, nx, ds, aw, ah):
    i = pl.program_id(0)
    a = jax.lax.rem(i, 3)
    y = x_ref[0]
    p_count = y.shape[1]
    c = jax.lax.broadcasted_iota(jnp.int32, (_OC, p_count), 0)
    p = jax.lax.broadcasted_iota(jnp.int32, (_OC, p_count), 1)
    sig = jax.nn.sigmoid(y)
    ex = jnp.exp(y)
    gx = (p % nx).astype(jnp.float32)
    gy = (p // nx).astype(jnp.float32)
    g = jnp.where(c == 0, gx, gy)
    anc_w = jnp.where(a == 0, aw[0], jnp.where(a == 1, aw[1], aw[2]))
    anc_h = jnp.where(a == 0, ah[0], jnp.where(a == 1, ah[1], ah[2]))
    anc = jnp.where(c == 2, anc_w, anc_h)
    xywh = jnp.where(c < 2, (sig + g) * ds, ex * anc)
    out = jnp.where(c < 4, xywh, sig)
    o_ref[0] = out.T


def _decode_scale(x, mask, ds):
    bs = x.shape[0]
    ny, nx = x.shape[-2:]
    npix = ny * nx
    xr = x.reshape(bs * 3, _OC, npix)
    anc = _ANCHORS[np.array(mask)]
    body = functools.partial(
        _decode_body, nx=nx, ds=ds,
        aw=tuple(float(v) for v in anc[:, 0]),
        ah=tuple(float(v) for v in anc[:, 1]))
    out = pl.pallas_call(
        body,
        grid=(bs * 3,),
        in_specs=[pl.BlockSpec((1, _OC, npix), lambda i: (i, 0, 0))],
        out_specs=pl.BlockSpec((1, npix, _OC), lambda i: (i, 0, 0)),
        out_shape=jax.ShapeDtypeStruct((bs * 3, npix, _OC), jnp.float32),
    )(xr)
    return out.reshape(bs, 3 * npix, _OC)


def kernel(x0, x1, x2):
    zs = [
        _decode_scale(x, m, d)
        for x, m, d in zip((x0, x1, x2), _ANCHOR_MASKS, _DOWNSAMPLE)
    ]
    return jnp.concatenate(zs, axis=1)
